# baseline (speedup 1.0000x reference)
import functools

import jax
import jax.numpy as jnp
from jax import lax
from jax.experimental import pallas as pl
from jax.experimental.pallas import tpu as pltpu
from jax.experimental.pallas import tpu_sc as plsc

NP = 50000
NC = 50000
E = 800000
DIN = 128
H = 64

_BR = 1000
_NB_HALF = 25

_NTILE = 16
_HALF = 25000
_PAD_ROWS = 25088
_FLUSH = _PAD_ROWS // _NTILE
_ZR = 98
_CHUNK = 128
_NCHUNK = E // _CHUNK
_CPT = -(-_NCHUNK // _NTILE)
_CP = 80
_NCHP = E // _CP
_JP = -(-_NCHP // (2 * _NTILE))


def _sc_mesh():
    return plsc.VectorSubcoreMesh(core_axis_name="c", subcore_axis_name="s")


def _fill(ref, rows, width, value):
    def body(i, _):
        for kk in range(width // 16):
            ref[i, pl.ds(kk * 16, 16)] = jnp.full((16,), value, jnp.float32)
        return 0
    lax.fori_loop(0, rows, body, 0)


def _zero_acc(acc, zbuf, tid):
    for z in range(_FLUSH // _ZR):
        pltpu.sync_copy(zbuf, acc.at[pl.ds(tid * _FLUSH + z * _ZR, _ZR)])


def _clamp_rows(idx_v, rows_v, lo, lanes):
    for jv in range(_CHUNK // 16):
        v = idx_v[pl.ds(jv * 16, 16)]
        r = v - lo
        ok = (r >= 0) & (r < _HALF)
        dummy = _HALF + lanes + (jv % 4) * 16
        rows_v[pl.ds(jv * 16, 16)] = jnp.where(ok, r, dummy)


def _flush(acc, out_hbm, core, tid):
    pltpu.sync_copy(acc.at[pl.ds(tid * _FLUSH, _FLUSH)],
                    out_hbm.at[core, pl.ds(tid * _FLUSH, _FLUSH)])


def _sc_counts(dst_fk, d2):
    @functools.partial(
        pl.kernel,
        out_type=(jax.ShapeDtypeStruct((2, _PAD_ROWS, 16), jnp.float32),
                  jax.ShapeDtypeStruct((2, _PAD_ROWS, 16), jnp.float32)),
        mesh=_sc_mesh(),
        compiler_params=pltpu.CompilerParams(use_tc_tiling_on_sc=False),
        scratch_types=[
            pltpu.VMEM((_CHUNK,), jnp.int32),
            pltpu.VMEM((_CHUNK,), jnp.int32),
            pltpu.VMEM((_CHUNK,), jnp.int32),
            pltpu.VMEM((_CHUNK,), jnp.int32),
            pltpu.VMEM((_CHUNK, 16), jnp.float32),
            pltpu.VMEM((_ZR, 16), jnp.float32),
            pltpu.VMEM_SHARED((_PAD_ROWS + 8, 16), jnp.float32),
            pltpu.SemaphoreType.DMA,
            pltpu.SemaphoreType.DMA,
            pltpu.SemaphoreType.DMA,
            pltpu.SemaphoreType.DMA,
        ],
    )
    def k(dfk_hbm, drev_hbm, out_fk, out_rev, ix0, ix1, ro0, ro1, ones_v,
          zbuf_v, acc, si0, si1, ss0, ss1):
        core = lax.axis_index("c")
        tid = lax.axis_index("s")
        lo = core * _HALF
        lanes = lax.iota(jnp.int32, 16)
        idx = (ix0, ix1)
        rows = (ro0, ro1)
        sem_i = (si0, si1)
        sem_s = (ss0, ss1)
        _fill(ones_v, _CHUNK, 16, 1.0)
        _fill(zbuf_v, _ZR, 16, 0.0)

        for src_hbm, out_hbm in ((dfk_hbm, out_fk), (drev_hbm, out_rev)):
            _zero_acc(acc, zbuf_v, tid)
            plsc.subcore_barrier()

            for b in range(2):
                c0 = b * _NTILE + tid
                pltpu.async_copy(src_hbm.at[pl.ds(c0 * _CHUNK, _CHUNK)],
                                 idx[b], sem_i[b])

            def body(j, _):
                for b in range(2):
                    c = (2 * j + b) * _NTILE + tid
                    cn = c + 2 * _NTILE

                    @pl.when(c < _NCHUNK)
                    def _():
                        pltpu.make_async_copy(
                            src_hbm.at[pl.ds(c * _CHUNK, _CHUNK)], idx[b],
                            sem_i[b]).wait()

                        @pl.when(j > 0)
                        def _():
                            pltpu.make_async_copy(
                                ones_v, acc.at[rows[b]], sem_s[b]).wait()
                        _clamp_rows(idx[b], rows[b], lo, lanes)
                        pltpu.async_copy(ones_v, acc.at[rows[b]], sem_s[b],
                                         add=True)

                        @pl.when(cn < _NCHUNK)
                        def _():
                            pltpu.async_copy(
                                src_hbm.at[pl.ds(cn * _CHUNK, _CHUNK)],
                                idx[b], sem_i[b])

                        @pl.when(cn >= _NCHUNK)
                        def _():
                            pltpu.make_async_copy(
                                ones_v, acc.at[rows[b]], sem_s[b]).wait()
                return 0

            lax.fori_loop(0, _CPT // 2 + 1, body, 0)
            plsc.subcore_barrier()
            _flush(acc, out_hbm, core, tid)
            plsc.subcore_barrier()

    return k(dst_fk, d2)


def _sc_sage(hp, s2, d2):
    @functools.partial(
        pl.kernel,
        out_type=jax.ShapeDtypeStruct((2, _PAD_ROWS, H), jnp.float32),
        mesh=_sc_mesh(),
        compiler_params=pltpu.CompilerParams(use_tc_tiling_on_sc=False),
        scratch_types=[
            pltpu.VMEM((_CHUNK,), jnp.int32),
            pltpu.VMEM((_CHUNK,), jnp.int32),
            pltpu.VMEM((_CHUNK,), jnp.int32),
            pltpu.VMEM((_CHUNK,), jnp.int32),
            pltpu.VMEM((_CHUNK,), jnp.int32),
            pltpu.VMEM((_CHUNK,), jnp.int32),
            pltpu.VMEM((_CHUNK, H), jnp.float32),
            pltpu.VMEM((_CHUNK, H), jnp.float32),
            pltpu.VMEM((_ZR, H), jnp.float32),
            pltpu.VMEM_SHARED((_PAD_ROWS + 8, H), jnp.float32),
            pltpu.SemaphoreType.DMA,
            pltpu.SemaphoreType.DMA,
            pltpu.SemaphoreType.DMA,
            pltpu.SemaphoreType.DMA,
            pltpu.SemaphoreType.DMA,
            pltpu.SemaphoreType.DMA,
        ],
    )
    def k(hp_hbm, s2_hbm, d2_hbm, out_hbm, si0, si1, di0, di1, ro0, ro1,
          rb0, rb1, zbuf_v, acc, gi0, gi1, gg0, gg1, gs0, gs1):
        core = lax.axis_index("c")
        tid = lax.axis_index("s")
        lo = core * _HALF
        lanes = lax.iota(jnp.int32, 16)
        sidx = (si0, si1)
        didx = (di0, di1)
        rows = (ro0, ro1)
        rbuf = (rb0, rb1)
        sem_i = (gi0, gi1)
        sem_g = (gg0, gg1)
        sem_s = (gs0, gs1)
        _fill(zbuf_v, _ZR, H, 0.0)
        _zero_acc(acc, zbuf_v, tid)
        plsc.subcore_barrier()

        for b in range(2):
            c0 = b * _NTILE + tid
            pltpu.sync_copy(s2_hbm.at[pl.ds(c0 * _CHUNK, _CHUNK)], sidx[b])
            pltpu.sync_copy(d2_hbm.at[pl.ds(c0 * _CHUNK, _CHUNK)], didx[b])
            pltpu.async_copy(hp_hbm.at[sidx[b]], rbuf[b], sem_g[b])

        def body(j, _):
            for b in range(2):
                c = (2 * j + b) * _NTILE + tid
                cn = c + 2 * _NTILE

                @pl.when(c < _NCHUNK)
                def _():
                    pltpu.make_async_copy(hp_hbm.at[sidx[b]], rbuf[b],
                                          sem_g[b]).wait()
                    _clamp_rows(didx[b], rows[b], lo, lanes)
                    pltpu.async_copy(rbuf[b], acc.at[rows[b]], sem_s[b],
                                     add=True)

                    @pl.when(cn < _NCHUNK)
                    def _():
                        pltpu.async_copy(
                            s2_hbm.at[pl.ds(cn * _CHUNK, _CHUNK)], sidx[b],
                            sem_i[b])
                        pltpu.async_copy(
                            d2_hbm.at[pl.ds(cn * _CHUNK, _CHUNK)], didx[b],
                            sem_i[b])
                        pltpu.make_async_copy(
                            s2_hbm.at[pl.ds(cn * _CHUNK, _CHUNK)], sidx[b],
                            sem_i[b]).wait()
                        pltpu.make_async_copy(
                            d2_hbm.at[pl.ds(cn * _CHUNK, _CHUNK)], didx[b],
                            sem_i[b]).wait()
                        pltpu.make_async_copy(
                            rbuf[b], acc.at[rows[b]], sem_s[b]).wait()
                        pltpu.async_copy(hp_hbm.at[sidx[b]], rbuf[b],
                                         sem_g[b])

                    @pl.when(cn >= _NCHUNK)
                    def _():
                        pltpu.make_async_copy(
                            rbuf[b], acc.at[rows[b]], sem_s[b]).wait()
            return 0

        lax.fori_loop(0, _CPT // 2 + 1, body, 0)
        plsc.subcore_barrier()
        _flush(acc, out_hbm, core, tid)

    return k(hp, s2, d2)


def _rsqrt_vec(x):
    i = lax.bitcast_convert_type(x, jnp.int32)
    i = jnp.int32(0x5F3759DF) - (i >> 1)
    y = lax.bitcast_convert_type(i, jnp.float32)
    for _ in range(2):
        y = y * (1.5 - 0.5 * x * y * y)
    return y


_GDN = lax.GatherDimensionNumbers(
    offset_dims=(), collapsed_slice_dims=(0,), start_index_map=(0,))


def _lane_bcast_sum(v, lanes):
    for sh in (8, 4, 2, 1):
        idx = (lanes ^ sh).reshape(16, 1)
        v = v + lax.gather(v, idx, _GDN, (1,),
                           mode=lax.GatherScatterMode.PROMISE_IN_BOUNDS)
    return v


def _clamp_rows_p(idx_v, rows_v, lo, lanes):
    for jv in range(_CP // 16):
        v = idx_v[pl.ds(jv * 16, 16)]
        r = v - lo
        ok = (r >= 0) & (r < _HALF)
        dummy = _HALF + lanes + (jv % 4) * 16
        rows_v[pl.ds(jv * 16, 16)] = jnp.where(ok, r, dummy)


def _sc_prmp(hc, negpred, src, dst):
    @functools.partial(
        pl.kernel,
        out_type=jax.ShapeDtypeStruct((2, _PAD_ROWS, H), jnp.float32),
        mesh=_sc_mesh(),
        compiler_params=pltpu.CompilerParams(use_tc_tiling_on_sc=False),
        scratch_types=[
            pltpu.VMEM((_CP,), jnp.int32),
            pltpu.VMEM((_CP,), jnp.int32),
            pltpu.VMEM((_CP,), jnp.int32),
            pltpu.VMEM((_CP,), jnp.int32),
            pltpu.VMEM((_CP,), jnp.int32),
            pltpu.VMEM((_CP,), jnp.int32),
            pltpu.VMEM((_CP, H), jnp.float32),
            pltpu.VMEM((_CP, H), jnp.float32),
            pltpu.VMEM((_CP, H), jnp.float32),
            pltpu.VMEM((_CP, H), jnp.float32),
            pltpu.VMEM_SHARED((_PAD_ROWS, H), jnp.float32),
            pltpu.SemaphoreType.DMA,
            pltpu.SemaphoreType.DMA,
            pltpu.SemaphoreType.DMA,
            pltpu.SemaphoreType.DMA,
            pltpu.SemaphoreType.DMA,
            pltpu.SemaphoreType.DMA,
            pltpu.SemaphoreType.DMA,
            pltpu.SemaphoreType.DMA,
        ],
    )
    def k(hc_hbm, npred_hbm, src_hbm, dst_hbm, out_hbm,
          si0, si1, di0, di1, ro0, ro1, xb0, xb1, mb0, mb1,
          acc, gi0, gi1, gx0, gx1, gp0, gp1, gs0, gs1):
        core = lax.axis_index("c")
        tid = lax.axis_index("s")
        lo = core * _HALF
        lanes = lax.iota(jnp.int32, 16)
        sidx = (si0, si1)
        didx = (di0, di1)
        rows = (ro0, ro1)
        xbuf = (xb0, xb1)
        mbuf = (mb0, mb1)
        sem_i = (gi0, gi1)
        sem_gx = (gx0, gx1)
        sem_gp = (gp0, gp1)
        sem_s = (gs0, gs1)

        _fill(mb0, 56, H, 0.0)
        for z in range(_FLUSH // 56):
            pltpu.sync_copy(mb0.at[pl.ds(0, 56)],
                            acc.at[pl.ds(tid * _FLUSH + z * 56, 56)])
        plsc.subcore_barrier()

        def make_edge_body(xb, mb):
            def edge_body(e, _):
                d = [xb[e, pl.ds(kk * 16, 16)] for kk in range(H // 16)]
                s = d[0] + d[1] + d[2] + d[3]
                q = d[0] * d[0] + d[1] * d[1] + d[2] * d[2] + d[3] * d[3]
                mean = _lane_bcast_sum(s, lanes) * (1.0 / H)
                var = _lane_bcast_sum(q, lanes) * (1.0 / H) - mean * mean
                scale = _rsqrt_vec(var + 1e-5)
                shift = mean * scale
                for kk in range(H // 16):
                    mb[e, pl.ds(kk * 16, 16)] = d[kk] * scale - shift
                return 0
            return edge_body

        for b in range(2):
            c0 = b * _NTILE + tid
            pltpu.sync_copy(src_hbm.at[pl.ds(c0 * _CP, _CP)], sidx[b])
            pltpu.sync_copy(dst_hbm.at[pl.ds(c0 * _CP, _CP)], didx[b])
            pltpu.async_copy(hc_hbm.at[sidx[b]], xbuf[b], sem_gx[b])

        def body(j, _):
            for b in range(2):
                c = (2 * j + b) * _NTILE + tid
                cn = c + 2 * _NTILE

                @pl.when(c < _NCHP)
                def _():
                    pltpu.make_async_copy(hc_hbm.at[sidx[b]], xbuf[b],
                                          sem_gx[b]).wait()
                    pltpu.async_copy(npred_hbm.at[didx[b]], xbuf[b],
                                     sem_gp[b], add=True)

                    @pl.when(j > 0)
                    def _():
                        pltpu.make_async_copy(
                            mbuf[b], acc.at[rows[b]], sem_s[b]).wait()
                    _clamp_rows_p(didx[b], rows[b], lo, lanes)
                    pltpu.make_async_copy(npred_hbm.at[didx[b]], xbuf[b],
                                          sem_gp[b]).wait()

                    @pl.when(cn < _NCHP)
                    def _():
                        pltpu.async_copy(
                            src_hbm.at[pl.ds(cn * _CP, _CP)], sidx[b],
                            sem_i[b])
                        pltpu.async_copy(
                            dst_hbm.at[pl.ds(cn * _CP, _CP)], didx[b],
                            sem_i[b])
                    lax.fori_loop(0, _CP, make_edge_body(xbuf[b], mbuf[b]),
                                  0, unroll=4)
                    pltpu.async_copy(mbuf[b], acc.at[rows[b]], sem_s[b],
                                     add=True)

                    @pl.when(cn < _NCHP)
                    def _():
                        pltpu.make_async_copy(
                            src_hbm.at[pl.ds(cn * _CP, _CP)], sidx[b],
                            sem_i[b]).wait()
                        pltpu.make_async_copy(
                            dst_hbm.at[pl.ds(cn * _CP, _CP)], didx[b],
                            sem_i[b]).wait()
                        pltpu.async_copy(hc_hbm.at[sidx[b]], xbuf[b],
                                         sem_gx[b])

                    @pl.when(cn >= _NCHP)
                    def _():
                        pltpu.make_async_copy(
                            mbuf[b], acc.at[rows[b]], sem_s[b]).wait()
            return 0

        lax.fori_loop(0, _JP, body, 0)
        plsc.subcore_barrier()
        _flush(acc, out_hbm, core, tid)

    return k(hc, negpred, src, dst)


def _mlp2_body(sign, x_ref, w1_ref, b1_ref, w2_ref, b2_ref, o_ref):
    h = jnp.maximum(
        jnp.dot(x_ref[...], w1_ref[...], preferred_element_type=jnp.float32)
        + b1_ref[...], 0.0)
    o_ref[...] = sign * (
        jnp.dot(h, w2_ref[...], preferred_element_type=jnp.float32)
        + b2_ref[...])


def _mlp2(x, W1, b1, W2, b2, sign=1.0):
    N, Din = x.shape
    H1 = W1.shape[1]
    H2 = W2.shape[1]
    return pl.pallas_call(
        functools.partial(_mlp2_body, sign),
        grid=(N // _BR,),
        in_specs=[
            pl.BlockSpec((_BR, Din), lambda i: (i, 0)),
            pl.BlockSpec((Din, H1), lambda i: (0, 0)),
            pl.BlockSpec((1, H1), lambda i: (0, 0)),
            pl.BlockSpec((H1, H2), lambda i: (0, 0)),
            pl.BlockSpec((1, H2), lambda i: (0, 0)),
        ],
        out_specs=pl.BlockSpec((_BR, H2), lambda i: (i, 0)),
        out_shape=jax.ShapeDtypeStruct((N, H2), jnp.float32),
    )(x, W1, b1.reshape(1, H1), W2, b2.reshape(1, H2))


def _ln_rows(h, g_ref, bln_ref):
    m = h.mean(-1, keepdims=True)
    v = ((h - m) ** 2).mean(-1, keepdims=True)
    return (h - m) / jnp.sqrt(v + 1e-5) * g_ref[...] + bln_ref[...]


def _padded_spec(w):
    return pl.BlockSpec((1, _BR, w),
                        lambda i: (i // _NB_HALF, i % _NB_HALF, 0))


def _upd_body(residual, hp_ref, s_ref, c_ref, wt_ref, wb_ref, b_ref,
              g_ref, bln_ref, mg_ref, mb_ref, o_ref):
    cnt = c_ref[0]
    s = s_ref[0]
    nonzero = cnt > 0.0
    aggr = jnp.where(nonzero,
                     s * mg_ref[...] / jnp.maximum(cnt, 1.0) + mb_ref[...],
                     0.0)
    h = (jnp.dot(hp_ref[...], wt_ref[...], preferred_element_type=jnp.float32)
         + jnp.dot(aggr, wb_ref[...], preferred_element_type=jnp.float32)
         + b_ref[...])
    h = jnp.maximum(_ln_rows(h, g_ref, bln_ref), 0.0)
    if residual:
        h = h + hp_ref[...]
    o_ref[...] = h


def _upd(hp, sums, cnt, Wt, Wb, b, g, bln, mg, mb, residual):
    N = hp.shape[0]
    return pl.pallas_call(
        functools.partial(_upd_body, residual),
        grid=(N // _BR,),
        in_specs=[
            pl.BlockSpec((_BR, H), lambda i: (i, 0)),
            _padded_spec(H),
            _padded_spec(1),
            pl.BlockSpec((H, H), lambda i: (0, 0)),
            pl.BlockSpec((H, H), lambda i: (0, 0)),
            pl.BlockSpec((1, H), lambda i: (0, 0)),
            pl.BlockSpec((1, H), lambda i: (0, 0)),
            pl.BlockSpec((1, H), lambda i: (0, 0)),
            pl.BlockSpec((1, H), lambda i: (0, 0)),
            pl.BlockSpec((1, H), lambda i: (0, 0)),
        ],
        out_specs=pl.BlockSpec((_BR, H), lambda i: (i, 0)),
        out_shape=jax.ShapeDtypeStruct((N, H), jnp.float32),
    )(hp, sums, cnt[:, :, :1], Wt, Wb, b.reshape(1, H), g.reshape(1, H),
      bln.reshape(1, H), mg.reshape(1, H), mb.reshape(1, H))


def _sage_body(residual, hc_ref, s_ref, c_ref, wl_ref, bl_ref, wr_ref,
               g_ref, bln_ref, o_ref):
    aggr = s_ref[0] / jnp.maximum(c_ref[0], 1.0)
    h = (jnp.dot(aggr, wl_ref[...], preferred_element_type=jnp.float32)
         + bl_ref[...]
         + jnp.dot(hc_ref[...], wr_ref[...], preferred_element_type=jnp.float32))
    h = jnp.maximum(_ln_rows(h, g_ref, bln_ref), 0.0)
    if residual:
        h = h + hc_ref[...]
    o_ref[...] = h


def _sage(hc, sums, cnt, Wl, bl, Wr, g, bln, residual):
    N = hc.shape[0]
    return pl.pallas_call(
        functools.partial(_sage_body, residual),
        grid=(N // _BR,),
        in_specs=[
            pl.BlockSpec((_BR, H), lambda i: (i, 0)),
            _padded_spec(H),
            _padded_spec(1),
            pl.BlockSpec((H, H), lambda i: (0, 0)),
            pl.BlockSpec((1, H), lambda i: (0, 0)),
            pl.BlockSpec((H, H), lambda i: (0, 0)),
            pl.BlockSpec((1, H), lambda i: (0, 0)),
            pl.BlockSpec((1, H), lambda i: (0, 0)),
        ],
        out_specs=pl.BlockSpec((_BR, H), lambda i: (i, 0)),
        out_shape=jax.ShapeDtypeStruct((N, H), jnp.float32),
    )(hc, sums, cnt[:, :, :1], Wl, bl.reshape(1, H), Wr, g.reshape(1, H),
      bln.reshape(1, H))


def kernel(x_parent, x_child, params, edge_index_fk, edge_index_rev):
    p = params
    src, dst = edge_index_fk[0], edge_index_fk[1]
    s2, d2 = edge_index_rev[0], edge_index_rev[1]

    cnt_fk, cnt_rev = _sc_counts(dst, d2)

    hp = _mlp2(x_parent, p['penc_W1'], p['penc_b1'], p['penc_W2'], p['penc_b2'])
    hc = _mlp2(x_child, p['cenc_W1'], p['cenc_b1'], p['cenc_W2'], p['cenc_b2'])

    for i in range(2):
        negpred = _mlp2(hp, p[f'l{i}_pred_W1'], p[f'l{i}_pred_b1'],
                        p[f'l{i}_pred_W2'], p[f'l{i}_pred_b2'], sign=-1.0)
        sums = _sc_prmp(hc, negpred, src, dst)
        Wt = p[f'l{i}_upd_W'][:H]
        Wb = p[f'l{i}_upd_W'][H:]
        new_hp = _upd(hp, sums, cnt_fk, Wt, Wb, p[f'l{i}_upd_b'],
                      p[f'l{i}_np_g'], p[f'l{i}_np_b'],
                      p[f'l{i}_msg_g'], p[f'l{i}_msg_b'], residual=(i > 0))
        sums2 = _sc_sage(hp, s2, d2)
        new_hc = _sage(hc, sums2, cnt_rev, p[f'l{i}_sage_Wl'],
                       p[f'l{i}_sage_bl'], p[f'l{i}_sage_Wr'],
                       p[f'l{i}_nc_g'], p[f'l{i}_nc_b'], residual=(i > 0))
        hp, hc = new_hp, new_hc

    out = _mlp2(hp, p['head_W1'], p['head_b1'], p['head_W2'], p['head_b2'])
    return out.squeeze(-1)


# speedup vs baseline: 2.3995x; 2.3995x over previous
import functools

import jax
import jax.numpy as jnp
from jax import lax
from jax.experimental import pallas as pl
from jax.experimental.pallas import tpu as pltpu
from jax.experimental.pallas import tpu_sc as plsc

NP = 50000
NC = 50000
E = 800000
DIN = 128
H = 64

_BR = 1000
_NB_HALF = 25

_NTILE = 16
_HALF = 25000
_PAD_ROWS = 25088
_FLUSH = _PAD_ROWS // _NTILE
_ZR = 98
_CHUNK = 128
_NCHUNK = E // _CHUNK
_CPT = -(-_NCHUNK // _NTILE)
_CP = 64
_NCHP = E // _CP
_JP = -(-_NCHP // (2 * _NTILE))


def _sc_mesh():
    return plsc.VectorSubcoreMesh(core_axis_name="c", subcore_axis_name="s")


def _fill(ref, rows, width, value):
    def body(i, _):
        for kk in range(width // 16):
            ref[i, pl.ds(kk * 16, 16)] = jnp.full((16,), value, jnp.float32)
        return 0
    lax.fori_loop(0, rows, body, 0)


def _zero_acc(acc, zbuf, tid):
    for z in range(_FLUSH // _ZR):
        pltpu.sync_copy(zbuf, acc.at[pl.ds(tid * _FLUSH + z * _ZR, _ZR)])


def _clamp_rows(idx_v, rows_v, lo, lanes):
    for jv in range(_CHUNK // 16):
        v = idx_v[pl.ds(jv * 16, 16)]
        r = v - lo
        ok = (r >= 0) & (r < _HALF)
        dummy = _HALF + lanes + (jv % 4) * 16
        rows_v[pl.ds(jv * 16, 16)] = jnp.where(ok, r, dummy)


def _flush(acc, out_hbm, core, tid):
    pltpu.sync_copy(acc.at[pl.ds(tid * _FLUSH, _FLUSH)],
                    out_hbm.at[core, pl.ds(tid * _FLUSH, _FLUSH)])


def _sc_counts(dst_fk, d2):
    @functools.partial(
        pl.kernel,
        out_type=(jax.ShapeDtypeStruct((2, _PAD_ROWS, 16), jnp.float32),
                  jax.ShapeDtypeStruct((2, _PAD_ROWS, 16), jnp.float32)),
        mesh=_sc_mesh(),
        compiler_params=pltpu.CompilerParams(use_tc_tiling_on_sc=False),
        scratch_types=[
            pltpu.VMEM((_CHUNK,), jnp.int32),
            pltpu.VMEM((_CHUNK,), jnp.int32),
            pltpu.VMEM((_CHUNK,), jnp.int32),
            pltpu.VMEM((_CHUNK,), jnp.int32),
            pltpu.VMEM((_CHUNK, 16), jnp.float32),
            pltpu.VMEM((_ZR, 16), jnp.float32),
            pltpu.VMEM_SHARED((_PAD_ROWS + 8, 16), jnp.float32),
            pltpu.SemaphoreType.DMA,
            pltpu.SemaphoreType.DMA,
            pltpu.SemaphoreType.DMA,
            pltpu.SemaphoreType.DMA,
        ],
    )
    def k(dfk_hbm, drev_hbm, out_fk, out_rev, ix0, ix1, ro0, ro1, ones_v,
          zbuf_v, acc, si0, si1, ss0, ss1):
        core = lax.axis_index("c")
        tid = lax.axis_index("s")
        lo = core * _HALF
        lanes = lax.iota(jnp.int32, 16)
        idx = (ix0, ix1)
        rows = (ro0, ro1)
        sem_i = (si0, si1)
        sem_s = (ss0, ss1)
        _fill(ones_v, _CHUNK, 16, 1.0)
        _fill(zbuf_v, _ZR, 16, 0.0)

        for src_hbm, out_hbm in ((dfk_hbm, out_fk), (drev_hbm, out_rev)):
            _zero_acc(acc, zbuf_v, tid)
            plsc.subcore_barrier()

            for b in range(2):
                c0 = b * _NTILE + tid
                pltpu.async_copy(src_hbm.at[pl.ds(c0 * _CHUNK, _CHUNK)],
                                 idx[b], sem_i[b])

            def body(j, _):
                for b in range(2):
                    c = (2 * j + b) * _NTILE + tid
                    cn = c + 2 * _NTILE

                    @pl.when(c < _NCHUNK)
                    def _():
                        pltpu.make_async_copy(
                            src_hbm.at[pl.ds(c * _CHUNK, _CHUNK)], idx[b],
                            sem_i[b]).wait()

                        @pl.when(j > 0)
                        def _():
                            pltpu.make_async_copy(
                                ones_v, acc.at[rows[b]], sem_s[b]).wait()
                        _clamp_rows(idx[b], rows[b], lo, lanes)
                        pltpu.async_copy(ones_v, acc.at[rows[b]], sem_s[b],
                                         add=True)

                        @pl.when(cn < _NCHUNK)
                        def _():
                            pltpu.async_copy(
                                src_hbm.at[pl.ds(cn * _CHUNK, _CHUNK)],
                                idx[b], sem_i[b])

                        @pl.when(cn >= _NCHUNK)
                        def _():
                            pltpu.make_async_copy(
                                ones_v, acc.at[rows[b]], sem_s[b]).wait()
                return 0

            lax.fori_loop(0, _CPT // 2 + 1, body, 0)
            plsc.subcore_barrier()
            _flush(acc, out_hbm, core, tid)
            plsc.subcore_barrier()

    return k(dst_fk, d2)


def _sc_sage(hp, s2, d2):
    @functools.partial(
        pl.kernel,
        out_type=jax.ShapeDtypeStruct((2, _PAD_ROWS, H), jnp.float32),
        mesh=_sc_mesh(),
        compiler_params=pltpu.CompilerParams(use_tc_tiling_on_sc=False),
        scratch_types=[
            pltpu.VMEM((_CHUNK,), jnp.int32),
            pltpu.VMEM((_CHUNK,), jnp.int32),
            pltpu.VMEM((_CHUNK,), jnp.int32),
            pltpu.VMEM((_CHUNK,), jnp.int32),
            pltpu.VMEM((_CHUNK,), jnp.int32),
            pltpu.VMEM((_CHUNK,), jnp.int32),
            pltpu.VMEM((_CHUNK, H), jnp.float32),
            pltpu.VMEM((_CHUNK, H), jnp.float32),
            pltpu.VMEM((_ZR, H), jnp.float32),
            pltpu.VMEM_SHARED((_PAD_ROWS + 8, H), jnp.float32),
            pltpu.SemaphoreType.DMA,
            pltpu.SemaphoreType.DMA,
            pltpu.SemaphoreType.DMA,
            pltpu.SemaphoreType.DMA,
            pltpu.SemaphoreType.DMA,
            pltpu.SemaphoreType.DMA,
        ],
    )
    def k(hp_hbm, s2_hbm, d2_hbm, out_hbm, si0, si1, di0, di1, ro0, ro1,
          rb0, rb1, zbuf_v, acc, gi0, gi1, gg0, gg1, gs0, gs1):
        core = lax.axis_index("c")
        tid = lax.axis_index("s")
        lo = core * _HALF
        lanes = lax.iota(jnp.int32, 16)
        sidx = (si0, si1)
        didx = (di0, di1)
        rows = (ro0, ro1)
        rbuf = (rb0, rb1)
        sem_i = (gi0, gi1)
        sem_g = (gg0, gg1)
        sem_s = (gs0, gs1)
        _fill(zbuf_v, _ZR, H, 0.0)
        _zero_acc(acc, zbuf_v, tid)
        plsc.subcore_barrier()

        for b in range(2):
            c0 = b * _NTILE + tid
            pltpu.sync_copy(s2_hbm.at[pl.ds(c0 * _CHUNK, _CHUNK)], sidx[b])
            pltpu.sync_copy(d2_hbm.at[pl.ds(c0 * _CHUNK, _CHUNK)], didx[b])
            pltpu.async_copy(hp_hbm.at[sidx[b]], rbuf[b], sem_g[b])

        def body(j, _):
            for b in range(2):
                c = (2 * j + b) * _NTILE + tid
                cn = c + 2 * _NTILE

                @pl.when(c < _NCHUNK)
                def _():
                    pltpu.make_async_copy(hp_hbm.at[sidx[b]], rbuf[b],
                                          sem_g[b]).wait()
                    _clamp_rows(didx[b], rows[b], lo, lanes)
                    pltpu.async_copy(rbuf[b], acc.at[rows[b]], sem_s[b],
                                     add=True)

                    @pl.when(cn < _NCHUNK)
                    def _():
                        pltpu.async_copy(
                            s2_hbm.at[pl.ds(cn * _CHUNK, _CHUNK)], sidx[b],
                            sem_i[b])
                        pltpu.async_copy(
                            d2_hbm.at[pl.ds(cn * _CHUNK, _CHUNK)], didx[b],
                            sem_i[b])
                        pltpu.make_async_copy(
                            s2_hbm.at[pl.ds(cn * _CHUNK, _CHUNK)], sidx[b],
                            sem_i[b]).wait()
                        pltpu.make_async_copy(
                            d2_hbm.at[pl.ds(cn * _CHUNK, _CHUNK)], didx[b],
                            sem_i[b]).wait()
                        pltpu.make_async_copy(
                            rbuf[b], acc.at[rows[b]], sem_s[b]).wait()
                        pltpu.async_copy(hp_hbm.at[sidx[b]], rbuf[b],
                                         sem_g[b])

                    @pl.when(cn >= _NCHUNK)
                    def _():
                        pltpu.make_async_copy(
                            rbuf[b], acc.at[rows[b]], sem_s[b]).wait()
            return 0

        lax.fori_loop(0, _CPT // 2 + 1, body, 0)
        plsc.subcore_barrier()
        _flush(acc, out_hbm, core, tid)

    return k(hp, s2, d2)


def _rsqrt_vec(x):
    i = lax.bitcast_convert_type(x, jnp.int32)
    i = jnp.int32(0x5F3759DF) - (i >> 1)
    y = lax.bitcast_convert_type(i, jnp.float32)
    for _ in range(2):
        y = y * (1.5 - 0.5 * x * y * y)
    return y


_GDN = lax.GatherDimensionNumbers(
    offset_dims=(), collapsed_slice_dims=(0,), start_index_map=(0,))


def _lane_bcast_sum(v, lanes):
    for sh in (8, 4, 2, 1):
        idx = (lanes ^ sh).reshape(16, 1)
        v = v + lax.gather(v, idx, _GDN, (1,),
                           mode=lax.GatherScatterMode.PROMISE_IN_BOUNDS)
    return v


def _clamp_rows_p(idx_v, rows_v, lo, lanes):
    for jv in range(_CP // 16):
        v = idx_v[pl.ds(jv * 16, 16)]
        r = v - lo
        ok = (r >= 0) & (r < _HALF)
        dummy = _HALF + lanes + (jv % 4) * 16
        rows_v[pl.ds(jv * 16, 16)] = jnp.where(ok, r, dummy)


def _sc_prmp(hc, pred, src, dst):
    @functools.partial(
        pl.kernel,
        out_type=jax.ShapeDtypeStruct((2, _PAD_ROWS, H), jnp.float32),
        mesh=_sc_mesh(),
        compiler_params=pltpu.CompilerParams(use_tc_tiling_on_sc=False),
        scratch_types=[
            pltpu.VMEM((_CP,), jnp.int32),
            pltpu.VMEM((_CP,), jnp.int32),
            pltpu.VMEM((_CP,), jnp.int32),
            pltpu.VMEM((_CP,), jnp.int32),
            pltpu.VMEM((_CP,), jnp.int32),
            pltpu.VMEM((_CP,), jnp.int32),
            pltpu.VMEM((_CP, H), jnp.float32),
            pltpu.VMEM((_CP, H), jnp.float32),
            pltpu.VMEM((_CP, H), jnp.float32),
            pltpu.VMEM((_CP, H), jnp.float32),
            pltpu.VMEM((_CP, H), jnp.float32),
            pltpu.VMEM((_CP, H), jnp.float32),
            pltpu.VMEM_SHARED((_PAD_ROWS, H), jnp.float32),
            pltpu.SemaphoreType.DMA,
            pltpu.SemaphoreType.DMA,
            pltpu.SemaphoreType.DMA,
            pltpu.SemaphoreType.DMA,
            pltpu.SemaphoreType.DMA,
            pltpu.SemaphoreType.DMA,
            pltpu.SemaphoreType.DMA,
            pltpu.SemaphoreType.DMA,
        ],
    )
    def k(hc_hbm, pred_hbm, src_hbm, dst_hbm, out_hbm,
          si0, si1, di0, di1, ro0, ro1, xb0, xb1, pb0, pb1, mb0, mb1,
          acc, gi0, gi1, gx0, gx1, gp0, gp1, gs0, gs1):
        core = lax.axis_index("c")
        tid = lax.axis_index("s")
        lo = core * _HALF
        lanes = lax.iota(jnp.int32, 16)
        sidx = (si0, si1)
        didx = (di0, di1)
        rows = (ro0, ro1)
        xbuf = (xb0, xb1)
        pbuf = (pb0, pb1)
        mbuf = (mb0, mb1)
        sem_i = (gi0, gi1)
        sem_gx = (gx0, gx1)
        sem_gp = (gp0, gp1)
        sem_s = (gs0, gs1)

        _fill(mb0, 56, H, 0.0)
        for z in range(_FLUSH // 56):
            pltpu.sync_copy(mb0.at[pl.ds(0, 56)],
                            acc.at[pl.ds(tid * _FLUSH + z * 56, 56)])
        plsc.subcore_barrier()

        def make_edge_body(xb, pb, mb):
            def edge_body(e):
                d = [xb[e, pl.ds(kk * 16, 16)] - pb[e, pl.ds(kk * 16, 16)]
                     for kk in range(H // 16)]
                s = d[0] + d[1] + d[2] + d[3]
                q = d[0] * d[0] + d[1] * d[1] + d[2] * d[2] + d[3] * d[3]
                mean = _lane_bcast_sum(s, lanes) * (1.0 / H)
                var = _lane_bcast_sum(q, lanes) * (1.0 / H) - mean * mean
                scale = _rsqrt_vec(var + 1e-5)
                shift = mean * scale
                for kk in range(H // 16):
                    mb[e, pl.ds(kk * 16, 16)] = d[kk] * scale - shift
            return edge_body

        for b in range(2):
            c0 = b * _NTILE + tid
            pltpu.sync_copy(src_hbm.at[pl.ds(c0 * _CP, _CP)], sidx[b])
            pltpu.sync_copy(dst_hbm.at[pl.ds(c0 * _CP, _CP)], didx[b])
            pltpu.async_copy(hc_hbm.at[sidx[b]], xbuf[b], sem_gx[b])
            pltpu.async_copy(pred_hbm.at[didx[b]], pbuf[b], sem_gp[b])

        def body(j, _):
            for b in range(2):
                c = (2 * j + b) * _NTILE + tid
                cn = c + 2 * _NTILE

                @pl.when(c < _NCHP)
                def _():
                    pltpu.make_async_copy(hc_hbm.at[sidx[b]], xbuf[b],
                                          sem_gx[b]).wait()
                    pltpu.make_async_copy(pred_hbm.at[didx[b]], pbuf[b],
                                          sem_gp[b]).wait()

                    @pl.when(j > 0)
                    def _():
                        pltpu.make_async_copy(
                            mbuf[b], acc.at[rows[b]], sem_s[b]).wait()
                    _clamp_rows_p(didx[b], rows[b], lo, lanes)

                    @pl.when(cn < _NCHP)
                    def _():
                        pltpu.async_copy(
                            src_hbm.at[pl.ds(cn * _CP, _CP)], sidx[b],
                            sem_i[b])
                        pltpu.async_copy(
                            dst_hbm.at[pl.ds(cn * _CP, _CP)], didx[b],
                            sem_i[b])
                    plsc.parallel_loop(0, _CP, 1, unroll=4)(
                        make_edge_body(xbuf[b], pbuf[b], mbuf[b]))
                    pltpu.async_copy(mbuf[b], acc.at[rows[b]], sem_s[b],
                                     add=True)

                    @pl.when(cn < _NCHP)
                    def _():
                        pltpu.make_async_copy(
                            src_hbm.at[pl.ds(cn * _CP, _CP)], sidx[b],
                            sem_i[b]).wait()
                        pltpu.make_async_copy(
                            dst_hbm.at[pl.ds(cn * _CP, _CP)], didx[b],
                            sem_i[b]).wait()
                        pltpu.async_copy(hc_hbm.at[sidx[b]], xbuf[b],
                                         sem_gx[b])
                        pltpu.async_copy(pred_hbm.at[didx[b]], pbuf[b],
                                         sem_gp[b])

                    @pl.when(cn >= _NCHP)
                    def _():
                        pltpu.make_async_copy(
                            mbuf[b], acc.at[rows[b]], sem_s[b]).wait()
            return 0

        lax.fori_loop(0, _JP, body, 0)
        plsc.subcore_barrier()
        _flush(acc, out_hbm, core, tid)

    return k(hc, pred, src, dst)


def _mlp2_body(sign, x_ref, w1_ref, b1_ref, w2_ref, b2_ref, o_ref):
    h = jnp.maximum(
        jnp.dot(x_ref[...], w1_ref[...], preferred_element_type=jnp.float32)
        + b1_ref[...], 0.0)
    o_ref[...] = sign * (
        jnp.dot(h, w2_ref[...], preferred_element_type=jnp.float32)
        + b2_ref[...])


def _mlp2(x, W1, b1, W2, b2, sign=1.0):
    N, Din = x.shape
    H1 = W1.shape[1]
    H2 = W2.shape[1]
    return pl.pallas_call(
        functools.partial(_mlp2_body, sign),
        grid=(N // _BR,),
        in_specs=[
            pl.BlockSpec((_BR, Din), lambda i: (i, 0)),
            pl.BlockSpec((Din, H1), lambda i: (0, 0)),
            pl.BlockSpec((1, H1), lambda i: (0, 0)),
            pl.BlockSpec((H1, H2), lambda i: (0, 0)),
            pl.BlockSpec((1, H2), lambda i: (0, 0)),
        ],
        out_specs=pl.BlockSpec((_BR, H2), lambda i: (i, 0)),
        out_shape=jax.ShapeDtypeStruct((N, H2), jnp.float32),
    )(x, W1, b1.reshape(1, H1), W2, b2.reshape(1, H2))


def _ln_rows(h, g_ref, bln_ref):
    m = h.mean(-1, keepdims=True)
    v = ((h - m) ** 2).mean(-1, keepdims=True)
    return (h - m) / jnp.sqrt(v + 1e-5) * g_ref[...] + bln_ref[...]


def _padded_spec(w):
    return pl.BlockSpec((1, _BR, w),
                        lambda i: (i // _NB_HALF, i % _NB_HALF, 0))


def _upd_body(residual, hp_ref, s_ref, c_ref, wt_ref, wb_ref, b_ref,
              g_ref, bln_ref, mg_ref, mb_ref, o_ref):
    cnt = c_ref[0]
    s = s_ref[0]
    nonzero = cnt > 0.0
    aggr = jnp.where(nonzero,
                     s * mg_ref[...] / jnp.maximum(cnt, 1.0) + mb_ref[...],
                     0.0)
    h = (jnp.dot(hp_ref[...], wt_ref[...], preferred_element_type=jnp.float32)
         + jnp.dot(aggr, wb_ref[...], preferred_element_type=jnp.float32)
         + b_ref[...])
    h = jnp.maximum(_ln_rows(h, g_ref, bln_ref), 0.0)
    if residual:
        h = h + hp_ref[...]
    o_ref[...] = h


def _upd(hp, sums, cnt, Wt, Wb, b, g, bln, mg, mb, residual):
    N = hp.shape[0]
    return pl.pallas_call(
        functools.partial(_upd_body, residual),
        grid=(N // _BR,),
        in_specs=[
            pl.BlockSpec((_BR, H), lambda i: (i, 0)),
            _padded_spec(H),
            _padded_spec(1),
            pl.BlockSpec((H, H), lambda i: (0, 0)),
            pl.BlockSpec((H, H), lambda i: (0, 0)),
            pl.BlockSpec((1, H), lambda i: (0, 0)),
            pl.BlockSpec((1, H), lambda i: (0, 0)),
            pl.BlockSpec((1, H), lambda i: (0, 0)),
            pl.BlockSpec((1, H), lambda i: (0, 0)),
            pl.BlockSpec((1, H), lambda i: (0, 0)),
        ],
        out_specs=pl.BlockSpec((_BR, H), lambda i: (i, 0)),
        out_shape=jax.ShapeDtypeStruct((N, H), jnp.float32),
    )(hp, sums, cnt[:, :, :1], Wt, Wb, b.reshape(1, H), g.reshape(1, H),
      bln.reshape(1, H), mg.reshape(1, H), mb.reshape(1, H))


def _sage_body(residual, hc_ref, s_ref, c_ref, wl_ref, bl_ref, wr_ref,
               g_ref, bln_ref, o_ref):
    aggr = s_ref[0] / jnp.maximum(c_ref[0], 1.0)
    h = (jnp.dot(aggr, wl_ref[...], preferred_element_type=jnp.float32)
         + bl_ref[...]
         + jnp.dot(hc_ref[...], wr_ref[...], preferred_element_type=jnp.float32))
    h = jnp.maximum(_ln_rows(h, g_ref, bln_ref), 0.0)
    if residual:
        h = h + hc_ref[...]
    o_ref[...] = h


def _sage(hc, sums, cnt, Wl, bl, Wr, g, bln, residual):
    N = hc.shape[0]
    return pl.pallas_call(
        functools.partial(_sage_body, residual),
        grid=(N // _BR,),
        in_specs=[
            pl.BlockSpec((_BR, H), lambda i: (i, 0)),
            _padded_spec(H),
            _padded_spec(1),
            pl.BlockSpec((H, H), lambda i: (0, 0)),
            pl.BlockSpec((1, H), lambda i: (0, 0)),
            pl.BlockSpec((H, H), lambda i: (0, 0)),
            pl.BlockSpec((1, H), lambda i: (0, 0)),
            pl.BlockSpec((1, H), lambda i: (0, 0)),
        ],
        out_specs=pl.BlockSpec((_BR, H), lambda i: (i, 0)),
        out_shape=jax.ShapeDtypeStruct((N, H), jnp.float32),
    )(hc, sums, cnt[:, :, :1], Wl, bl.reshape(1, H), Wr, g.reshape(1, H),
      bln.reshape(1, H))


def kernel(x_parent, x_child, params, edge_index_fk, edge_index_rev):
    p = params
    src, dst = edge_index_fk[0], edge_index_fk[1]
    s2, d2 = edge_index_rev[0], edge_index_rev[1]

    cnt_fk, cnt_rev = _sc_counts(dst, d2)

    hp = _mlp2(x_parent, p['penc_W1'], p['penc_b1'], p['penc_W2'], p['penc_b2'])
    hc = _mlp2(x_child, p['cenc_W1'], p['cenc_b1'], p['cenc_W2'], p['cenc_b2'])

    for i in range(2):
        pred = _mlp2(hp, p[f'l{i}_pred_W1'], p[f'l{i}_pred_b1'],
                     p[f'l{i}_pred_W2'], p[f'l{i}_pred_b2'])
        sums = _sc_prmp(hc, pred, src, dst)
        Wt = p[f'l{i}_upd_W'][:H]
        Wb = p[f'l{i}_upd_W'][H:]
        new_hp = _upd(hp, sums, cnt_fk, Wt, Wb, p[f'l{i}_upd_b'],
                      p[f'l{i}_np_g'], p[f'l{i}_np_b'],
                      p[f'l{i}_msg_g'], p[f'l{i}_msg_b'], residual=(i > 0))
        sums2 = _sc_sage(hp, s2, d2)
        new_hc = _sage(hc, sums2, cnt_rev, p[f'l{i}_sage_Wl'],
                       p[f'l{i}_sage_bl'], p[f'l{i}_sage_Wr'],
                       p[f'l{i}_nc_g'], p[f'l{i}_nc_b'], residual=(i > 0))
        hp, hc = new_hp, new_hc

    out = _mlp2(hp, p['head_W1'], p['head_b1'], p['head_W2'], p['head_b2'])
    return out.squeeze(-1)
